# baseline (speedup 1.0000x reference)
import functools

import jax
import jax.numpy as jnp
from jax import lax
from jax.experimental import pallas as pl
from jax.experimental.pallas import tpu as pltpu
from jax.experimental.pallas import tpu_sc as plsc

_NODE = {"drug": 20000, "disease": 20000, "gene": 60000}
_ETS = [("drug", "targets", "gene"), ("gene", "assoc", "disease"),
        ("gene", "rev_targets", "drug"), ("disease", "rev_assoc", "gene")]
_D = 128
_NC, _NS, _L = 2, 16, 16
_B = 128
_AU = 256


def _ekey(s, r, d):
    return s + "_" + r + "_" + d


def _ranges(n_dst):
    n_ranges = 6 if n_dst > 32000 else 2
    rpt = -(-n_dst // (n_ranges * _NS * 8)) * 8
    return n_ranges, _NS * rpt, rpt


def _remap(dst_ref, out_ref, lo, rng, n=_B):
    for i in range(n // _L):
        v = dst_ref[pl.ds(i * _L, _L)]
        lv = v - lo
        ok = (lv >= 0) & (lv < rng)
        out_ref[pl.ds(i * _L, _L)] = jnp.where(ok, lv, rng + (v & 127))


@functools.lru_cache(maxsize=None)
def _part_kernel(n_dst, nb, E):
    n_ranges, rng, _ = _ranges(n_dst)
    G = -(-E // (2 * _AU)) + 24
    nb2 = nb // _NC
    mesh = plsc.VectorSubcoreMesh(core_axis_name="c", subcore_axis_name="s")

    def body(srcb, dstb, psrc, pdst, counts_out, *scr):
        counters, vsrc, vdst, stage_s2, stage_d2, cvec, srt_s, srt_d = scr
        SW = 2 * _AU
        c = lax.axis_index("c")
        s = lax.axis_index("s")
        iota = jnp.arange(_L, dtype=jnp.int32)

        @pl.when(s == 0)
        def _():
            for q in range(n_ranges):
                counters[q] = 0

        plsc.subcore_barrier()

        def flush_dma(q):
            grp = plsc.fetch_and_add(counters.at[q], 1, subcore_id=0)
            pltpu.sync_copy(stage_s2.at[pl.ds(q * SW, _AU)],
                            psrc.at[c, q, grp])
            pltpu.sync_copy(stage_d2.at[pl.ds(q * SW, _AU)],
                            pdst.at[c, q, grp])

        def blk(j, fills):
            pltpu.sync_copy(srcb.at[s, c * nb2 + j], vsrc)
            pltpu.sync_copy(dstb.at[s, c * nb2 + j], vdst)
            fills = list(fills)
            for i in range(_B // _L):
                sv = vsrc[pl.ds(i * _L, _L)]
                dv = vdst[pl.ds(i * _L, _L)]
                rid = jnp.zeros((_L,), jnp.int32)
                for q in range(1, n_ranges):
                    rid = rid + jnp.where(dv >= q * rng, 1, 0)
                ks, svs = plsc.sort_key_val(rid, sv)
                _, dvs = plsc.sort_key_val(rid, dv)
                srt_s[pl.ds(0, _L)] = svs
                srt_d[pl.ds(0, _L)] = dvs
                off = 0
                for q in range(n_ranges):
                    m = ks == q
                    pc = jnp.sum(jnp.where(m, 1, 0))
                    f = fills[q]
                    stage_s2[pl.ds(q * SW + f, _L)] = srt_s[pl.ds(off, _L)]
                    stage_d2[pl.ds(q * SW + f, _L)] = srt_d[pl.ds(off, _L)]
                    fills[q] = f + pc
                    off = off + pc
            for q in range(n_ranges):
                f = fills[q]

                @pl.when(f >= _AU)
                def _():
                    flush_dma(q)
                    for t in range(_B // _L):
                        stage_s2[pl.ds(q * SW + t * _L, _L)] = \
                            stage_s2[pl.ds(q * SW + _AU + t * _L, _L)]
                        stage_d2[pl.ds(q * SW + t * _L, _L)] = \
                            stage_d2[pl.ds(q * SW + _AU + t * _L, _L)]
                fills[q] = jnp.where(f >= _AU, f - _AU, f)
            return tuple(fills)

        zero = jnp.zeros((), jnp.int32)
        fills = lax.fori_loop(0, nb2, blk, (zero,) * n_ranges)
        pad_s = jnp.zeros((_L,), jnp.int32)
        pad_d = jnp.full((_L,), n_dst, jnp.int32)
        for q in range(n_ranges):
            f = fills[q]

            @pl.when(f > 0)
            def _():
                for t in range(_AU // _L):
                    stage_s2[pl.ds(q * SW + f + t * _L, _L)] = pad_s
                    stage_d2[pl.ds(q * SW + f + t * _L, _L)] = pad_d
                flush_dma(q)

        plsc.subcore_barrier()

        @pl.when(s == 0)
        def _():
            cv = jnp.zeros((_L,), jnp.int32)
            for q in range(n_ranges):
                cv = jnp.where(iota == q, counters[q], cv)
            cvec[...] = cv
            pltpu.sync_copy(cvec, counts_out.at[c])

    return pl.kernel(
        body,
        out_type=(
            jax.ShapeDtypeStruct((_NC, n_ranges, G, _AU), jnp.int32),
            jax.ShapeDtypeStruct((_NC, n_ranges, G, _AU), jnp.int32),
            jax.ShapeDtypeStruct((_NC, _L), jnp.int32),
        ),
        mesh=mesh,
        compiler_params=pltpu.CompilerParams(needs_layout_passes=False),
        scratch_types=(
            [pltpu.SMEM((_L,), jnp.int32),
             pltpu.VMEM((_B,), jnp.int32),
             pltpu.VMEM((_B,), jnp.int32),
             pltpu.VMEM((n_ranges * 2 * _AU,), jnp.int32),
             pltpu.VMEM((n_ranges * 2 * _AU,), jnp.int32),
             pltpu.VMEM((_L,), jnp.int32),
             pltpu.VMEM((2 * _L,), jnp.int32),
             pltpu.VMEM((2 * _L,), jnp.int32)]))


def _group_count(cnts_v, r, q):
    iota = jnp.arange(_L, dtype=jnp.int32)
    return jnp.max(jnp.where(iota == q, cnts_v[r], 0))


@functools.lru_cache(maxsize=None)
def _agg_kernel(n_src, n_dst, E):
    n_ranges, rng, rpt = _ranges(n_dst)
    mesh = plsc.VectorSubcoreMesh(core_axis_name="c", subcore_axis_name="s")

    def body(x_hbm, psrc, pdst, counts_hbm, zeros_hbm, out_hbm,
             acc, cnts_v, idxs, idxd, idxd2, rows, sem):
        c = lax.axis_index("c")
        s = lax.axis_index("s")
        pltpu.sync_copy(counts_hbm, cnts_v)

        def one_pass(q):
            lo = q * rng
            pltpu.sync_copy(zeros_hbm.at[pl.ds(0, rpt)],
                            acc.at[pl.ds(s * rpt, rpt)])
            plsc.subcore_barrier()
            for r in range(_NC):
                gq = _group_count(cnts_v, r, q)
                n_i = jnp.maximum(0, (gq - s + _NS - 1) // _NS)

                def gblk(i, carry):
                    base = (s + i * _NS) * _AU
                    pltpu.sync_copy(psrc.at[r, q, pl.ds(base, _AU)], idxs)
                    pltpu.sync_copy(pdst.at[r, q, pl.ds(base, _AU)], idxd)
                    _remap(idxd, idxd2, lo, rng, _AU)
                    pltpu.async_copy(x_hbm.at[idxs], rows, sem).wait()
                    pltpu.sync_copy(rows, acc.at[idxd2], add=True)
                    return carry

                lax.fori_loop(0, n_i, gblk, 0)
            plsc.subcore_barrier()
            pltpu.sync_copy(acc.at[pl.ds(s * rpt, rpt)],
                            out_hbm.at[pl.ds(lo + s * rpt, rpt)])
            plsc.subcore_barrier()

        for half in range(_NC):
            @pl.when(c == half)
            def _():
                for q in range(half, n_ranges, _NC):
                    one_pass(q)

    return pl.kernel(
        body,
        out_type=jax.ShapeDtypeStruct((n_ranges * rng, _D), jnp.float32),
        mesh=mesh,
        compiler_params=pltpu.CompilerParams(needs_layout_passes=False),
        scratch_types=[
            pltpu.VMEM_SHARED((rng + 128, _D), jnp.float32),
            pltpu.VMEM((_NC, _L), jnp.int32),
            pltpu.VMEM((_AU,), jnp.int32),
            pltpu.VMEM((_AU,), jnp.int32),
            pltpu.VMEM((_AU,), jnp.int32),
            pltpu.VMEM((_AU, _D), jnp.float32),
            pltpu.SemaphoreType.DMA,
        ])


@functools.lru_cache(maxsize=None)
def _cnt_kernel(n_dst, E):
    n_ranges, rng, rpt = _ranges(n_dst)
    mesh = plsc.VectorSubcoreMesh(core_axis_name="c", subcore_axis_name="s")

    def body(pdst, counts_hbm, zeros_hbm, ones_hbm, out_hbm,
             acc, cnts_v, idxd, idxd2, ones_v, sem):
        c = lax.axis_index("c")
        s = lax.axis_index("s")
        pltpu.sync_copy(counts_hbm, cnts_v)
        pltpu.sync_copy(ones_hbm, ones_v)

        def one_pass(q):
            lo = q * rng
            pltpu.sync_copy(zeros_hbm.at[pl.ds(0, rpt)],
                            acc.at[pl.ds(s * rpt, rpt)])
            plsc.subcore_barrier()
            for r in range(_NC):
                gq = _group_count(cnts_v, r, q)
                n_i = jnp.maximum(0, (gq - s + _NS - 1) // _NS)

                def gblk(i, carry):
                    base = (s + i * _NS) * _AU
                    pltpu.sync_copy(pdst.at[r, q, pl.ds(base, _AU)], idxd)
                    _remap(idxd, idxd2, lo, rng, _AU)
                    pltpu.sync_copy(ones_v, acc.at[idxd2], add=True)
                    return carry

                lax.fori_loop(0, n_i, gblk, 0)
            plsc.subcore_barrier()
            pltpu.sync_copy(acc.at[pl.ds(s * rpt, rpt)],
                            out_hbm.at[pl.ds(lo + s * rpt, rpt)])
            plsc.subcore_barrier()

        for half in range(_NC):
            @pl.when(c == half)
            def _():
                for q in range(half, n_ranges, _NC):
                    one_pass(q)

    return pl.kernel(
        body,
        out_type=jax.ShapeDtypeStruct((n_ranges * rng, _D), jnp.float32),
        mesh=mesh,
        compiler_params=pltpu.CompilerParams(needs_layout_passes=False),
        scratch_types=[
            pltpu.VMEM_SHARED((rng + 128, _D), jnp.float32),
            pltpu.VMEM((_NC, _L), jnp.int32),
            pltpu.VMEM((_AU,), jnp.int32),
            pltpu.VMEM((_AU,), jnp.int32),
            pltpu.VMEM((_AU, _D), jnp.float32),
            pltpu.SemaphoreType.DMA,
        ])


_R = 1000


@functools.lru_cache(maxsize=None)
def _combine_kernel(n, n_et):
    def body(*refs):
        aggs = refs[0:n_et]
        recips = refs[n_et:2 * n_et]
        x_ref = refs[2 * n_et]
        wls = refs[2 * n_et + 1:3 * n_et + 1]
        wr_ref = refs[3 * n_et + 1]
        b_ref = refs[3 * n_et + 2]
        out_ref = refs[3 * n_et + 3]
        acc = jnp.dot(x_ref[...], wr_ref[...],
                      preferred_element_type=jnp.float32) + b_ref[...]
        for a, r, w in zip(aggs, recips, wls):
            acc = acc + jnp.dot(a[...] * r[...], w[...],
                                preferred_element_type=jnp.float32)
        out_ref[...] = jnp.maximum(acc, 0.0)

    row_spec = pl.BlockSpec((_R, _D), lambda i: (i, 0))
    one_spec = pl.BlockSpec((_R, 1), lambda i: (i, 0))
    w_spec = pl.BlockSpec((_D, _D), lambda i: (0, 0))
    b_spec = pl.BlockSpec((1, _D), lambda i: (0, 0))
    in_specs = ([row_spec] * n_et + [one_spec] * n_et + [row_spec]
                + [w_spec] * n_et + [w_spec, b_spec])
    return pl.pallas_call(
        body,
        grid=(n // _R,),
        in_specs=in_specs,
        out_specs=row_spec,
        out_shape=jax.ShapeDtypeStruct((n, _D), jnp.float32),
    )


def _pad_blocks(a, nb, fill):
    total = _NS * nb * _B
    a = jnp.concatenate(
        [a, jnp.full((total - a.shape[0],), fill, jnp.int32)])
    return a.reshape(_NS, nb, _B)


def kernel(params, edges):
    parts = {}
    cnt_recip = {}
    for (s, r, d) in _ETS:
        k = _ekey(s, r, d)
        e = edges[k]
        E = e.shape[1]
        n_dst = _NODE[d]
        _, _, rpt = _ranges(n_dst)
        nb = -(-E // (_NS * _B))
        nb += nb % 2
        srcb = _pad_blocks(e[0], nb, 0)
        dstb = _pad_blocks(e[1], nb, n_dst)
        psrc, pdst, gcnt = _part_kernel(n_dst, nb, E)(srcb, dstb)
        G = psrc.shape[2]
        psrc = psrc.reshape(_NC, psrc.shape[1], G * _AU)
        pdst = pdst.reshape(_NC, pdst.shape[1], G * _AU)
        parts[k] = (psrc, pdst, gcnt, E)
        zeros = jnp.zeros((rpt, _D), jnp.float32)
        ones = jnp.ones((_AU, _D), jnp.float32)
        cnt = _cnt_kernel(n_dst, E)(pdst, gcnt, zeros, ones)[:n_dst, 0]
        cnt_recip[k] = (1.0 / jnp.maximum(cnt, 1.0)).reshape(n_dst, 1)

    x = {nt: params["emb"][nt] for nt in _NODE}
    for l in range(2):
        lp = params["l" + str(l)]
        aggs = {}
        for (s, r, d) in _ETS:
            k = _ekey(s, r, d)
            psrc, pdst, gcnt, E = parts[k]
            n_dst = _NODE[d]
            _, _, rpt = _ranges(n_dst)
            zeros = jnp.zeros((rpt, _D), jnp.float32)
            aggs[k] = _agg_kernel(_NODE[s], n_dst, E)(
                x[s], psrc, pdst, gcnt, zeros)
        new_x = {}
        for nt in _NODE:
            ets = [(s, r, d) for (s, r, d) in _ETS if d == nt]
            n_et = len(ets)
            ks = [_ekey(*et) for et in ets]
            wr = sum(lp[k]["W_r"] for k in ks) / n_et
            bb = (sum(lp[k]["b_l"] for k in ks) / n_et).reshape(1, _D)
            args = ([aggs[k] for k in ks]
                    + [cnt_recip[k] for k in ks]
                    + [x[nt]] + [lp[k]["W_l"] / n_et for k in ks]
                    + [wr, bb])
            new_x[nt] = _combine_kernel(_NODE[nt], n_et)(*args)
        x = new_x
    return (x["drug"], x["disease"], x["gene"])


# speedup vs baseline: 1.0033x; 1.0033x over previous
import functools

import jax
import jax.numpy as jnp
from jax import lax
from jax.experimental import pallas as pl
from jax.experimental.pallas import tpu as pltpu
from jax.experimental.pallas import tpu_sc as plsc

_NODE = {"drug": 20000, "disease": 20000, "gene": 60000}
_ETS = [("drug", "targets", "gene"), ("gene", "assoc", "disease"),
        ("gene", "rev_targets", "drug"), ("disease", "rev_assoc", "gene")]
_D = 128
_NC, _NS, _L = 2, 16, 16
_B = 128
_AU = 256


def _ekey(s, r, d):
    return s + "_" + r + "_" + d


def _ranges(n_dst):
    n_ranges = 6 if n_dst > 32000 else 2
    rpt = -(-n_dst // (n_ranges * _NS * 8)) * 8
    return n_ranges, _NS * rpt, rpt


def _remap(dst_ref, out_ref, lo, rng, n=_B):
    for i in range(n // _L):
        v = dst_ref[pl.ds(i * _L, _L)]
        lv = v - lo
        ok = (lv >= 0) & (lv < rng)
        out_ref[pl.ds(i * _L, _L)] = jnp.where(ok, lv, rng + (v & 127))


@functools.lru_cache(maxsize=None)
def _part_kernel(n_dst, nb, E):
    n_ranges, rng, _ = _ranges(n_dst)
    G = -(-E // (2 * _AU)) + 24
    nb2 = nb // _NC
    mesh = plsc.VectorSubcoreMesh(core_axis_name="c", subcore_axis_name="s")

    def body(srcb, dstb, psrc, pdst, counts_out, *scr):
        counters, vsrc, vdst, stage_s2, stage_d2, cvec, srt_s, srt_d = scr
        SW = 2 * _AU
        c = lax.axis_index("c")
        s = lax.axis_index("s")
        iota = jnp.arange(_L, dtype=jnp.int32)

        @pl.when(s == 0)
        def _():
            for q in range(n_ranges):
                counters[q] = 0

        plsc.subcore_barrier()

        def flush_dma(q):
            grp = plsc.fetch_and_add(counters.at[q], 1, subcore_id=0)
            pltpu.sync_copy(stage_s2.at[pl.ds(q * SW, _AU)],
                            psrc.at[c, q, grp])
            pltpu.sync_copy(stage_d2.at[pl.ds(q * SW, _AU)],
                            pdst.at[c, q, grp])

        def blk(j, fills):
            pltpu.sync_copy(srcb.at[s, c * nb2 + j], vsrc)
            pltpu.sync_copy(dstb.at[s, c * nb2 + j], vdst)
            fills = list(fills)
            for i in range(_B // _L):
                sv = vsrc[pl.ds(i * _L, _L)]
                dv = vdst[pl.ds(i * _L, _L)]
                rid = jnp.zeros((_L,), jnp.int32)
                for q in range(1, n_ranges):
                    rid = rid + jnp.where(dv >= q * rng, 1, 0)
                ks, svs = plsc.sort_key_val(rid, sv)
                _, dvs = plsc.sort_key_val(rid, dv)
                srt_s[pl.ds(0, _L)] = svs
                srt_d[pl.ds(0, _L)] = dvs
                off = 0
                for q in range(n_ranges):
                    m = ks == q
                    pc = jnp.sum(jnp.where(m, 1, 0))
                    f = fills[q]
                    stage_s2[pl.ds(q * SW + f, _L)] = srt_s[pl.ds(off, _L)]
                    stage_d2[pl.ds(q * SW + f, _L)] = srt_d[pl.ds(off, _L)]
                    fills[q] = f + pc
                    off = off + pc
            for q in range(n_ranges):
                f = fills[q]

                @pl.when(f >= _AU)
                def _():
                    flush_dma(q)
                    for t in range(_B // _L):
                        stage_s2[pl.ds(q * SW + t * _L, _L)] = \
                            stage_s2[pl.ds(q * SW + _AU + t * _L, _L)]
                        stage_d2[pl.ds(q * SW + t * _L, _L)] = \
                            stage_d2[pl.ds(q * SW + _AU + t * _L, _L)]
                fills[q] = jnp.where(f >= _AU, f - _AU, f)
            return tuple(fills)

        zero = jnp.zeros((), jnp.int32)
        fills = lax.fori_loop(0, nb2, blk, (zero,) * n_ranges)
        pad_s = jnp.zeros((_L,), jnp.int32)
        pad_d = jnp.full((_L,), n_dst, jnp.int32)
        for q in range(n_ranges):
            f = fills[q]

            @pl.when(f > 0)
            def _():
                for t in range(_AU // _L):
                    stage_s2[pl.ds(q * SW + f + t * _L, _L)] = pad_s
                    stage_d2[pl.ds(q * SW + f + t * _L, _L)] = pad_d
                flush_dma(q)

        plsc.subcore_barrier()

        @pl.when(s == 0)
        def _():
            cv = jnp.zeros((_L,), jnp.int32)
            for q in range(n_ranges):
                cv = jnp.where(iota == q, counters[q], cv)
            cvec[...] = cv
            pltpu.sync_copy(cvec, counts_out.at[c])

    return pl.kernel(
        body,
        out_type=(
            jax.ShapeDtypeStruct((_NC, n_ranges, G, _AU), jnp.int32),
            jax.ShapeDtypeStruct((_NC, n_ranges, G, _AU), jnp.int32),
            jax.ShapeDtypeStruct((_NC, _L), jnp.int32),
        ),
        mesh=mesh,
        compiler_params=pltpu.CompilerParams(needs_layout_passes=False),
        scratch_types=(
            [pltpu.SMEM((_L,), jnp.int32),
             pltpu.VMEM((_B,), jnp.int32),
             pltpu.VMEM((_B,), jnp.int32),
             pltpu.VMEM((n_ranges * 2 * _AU,), jnp.int32),
             pltpu.VMEM((n_ranges * 2 * _AU,), jnp.int32),
             pltpu.VMEM((_L,), jnp.int32),
             pltpu.VMEM((2 * _L,), jnp.int32),
             pltpu.VMEM((2 * _L,), jnp.int32)]))


def _group_count(cnts_v, r, q):
    iota = jnp.arange(_L, dtype=jnp.int32)
    return jnp.max(jnp.where(iota == q, cnts_v[r], 0))


@functools.lru_cache(maxsize=None)
def _agg_kernel(n_src, n_dst, E):
    n_ranges, rng, rpt = _ranges(n_dst)
    mesh = plsc.VectorSubcoreMesh(core_axis_name="c", subcore_axis_name="s")

    def body(x_hbm, psrc, pdst, counts_hbm, zeros_hbm, out_hbm,
             acc, cnts_v, idxs, idxd, idxd2, rows, sem, sem2):
        c = lax.axis_index("c")
        s = lax.axis_index("s")
        pltpu.sync_copy(counts_hbm, cnts_v)

        def one_pass(q):
            lo = q * rng
            pltpu.sync_copy(zeros_hbm.at[pl.ds(0, rpt)],
                            acc.at[pl.ds(s * rpt, rpt)])
            plsc.subcore_barrier()
            for r in range(_NC):
                gq = _group_count(cnts_v, r, q)
                n_i = jnp.maximum(0, (gq - s + _NS - 1) // _NS)

                def gblk(i, carry):
                    base = (s + i * _NS) * _AU
                    pltpu.sync_copy(psrc.at[r, q, pl.ds(base, _AU)], idxs)
                    pltpu.sync_copy(pdst.at[r, q, pl.ds(base, _AU)], idxd)
                    _remap(idxd, idxd2, lo, rng, _AU)
                    cp0 = pltpu.async_copy(
                        x_hbm.at[idxs.at[pl.ds(0, _B)]],
                        rows.at[pl.ds(0, _B)], sem)
                    cp1 = pltpu.async_copy(
                        x_hbm.at[idxs.at[pl.ds(_B, _B)]],
                        rows.at[pl.ds(_B, _B)], sem2)
                    cp0.wait()
                    pltpu.sync_copy(rows.at[pl.ds(0, _B)],
                                    acc.at[idxd2.at[pl.ds(0, _B)]], add=True)
                    cp1.wait()
                    pltpu.sync_copy(rows.at[pl.ds(_B, _B)],
                                    acc.at[idxd2.at[pl.ds(_B, _B)]], add=True)
                    return carry

                lax.fori_loop(0, n_i, gblk, 0)
            plsc.subcore_barrier()
            pltpu.sync_copy(acc.at[pl.ds(s * rpt, rpt)],
                            out_hbm.at[pl.ds(lo + s * rpt, rpt)])
            plsc.subcore_barrier()

        for half in range(_NC):
            @pl.when(c == half)
            def _():
                for q in range(half, n_ranges, _NC):
                    one_pass(q)

    return pl.kernel(
        body,
        out_type=jax.ShapeDtypeStruct((n_ranges * rng, _D), jnp.float32),
        mesh=mesh,
        compiler_params=pltpu.CompilerParams(needs_layout_passes=False),
        scratch_types=[
            pltpu.VMEM_SHARED((rng + 128, _D), jnp.float32),
            pltpu.VMEM((_NC, _L), jnp.int32),
            pltpu.VMEM((_AU,), jnp.int32),
            pltpu.VMEM((_AU,), jnp.int32),
            pltpu.VMEM((_AU,), jnp.int32),
            pltpu.VMEM((_AU, _D), jnp.float32),
            pltpu.SemaphoreType.DMA,
            pltpu.SemaphoreType.DMA,
        ])


@functools.lru_cache(maxsize=None)
def _cnt_kernel(n_dst, E):
    n_ranges, rng, rpt = _ranges(n_dst)
    mesh = plsc.VectorSubcoreMesh(core_axis_name="c", subcore_axis_name="s")

    def body(pdst, counts_hbm, zeros_hbm, ones_hbm, out_hbm,
             acc, cnts_v, idxd, idxd2, ones_v, sem):
        c = lax.axis_index("c")
        s = lax.axis_index("s")
        pltpu.sync_copy(counts_hbm, cnts_v)
        pltpu.sync_copy(ones_hbm, ones_v)

        def one_pass(q):
            lo = q * rng
            pltpu.sync_copy(zeros_hbm.at[pl.ds(0, rpt)],
                            acc.at[pl.ds(s * rpt, rpt)])
            plsc.subcore_barrier()
            for r in range(_NC):
                gq = _group_count(cnts_v, r, q)
                n_i = jnp.maximum(0, (gq - s + _NS - 1) // _NS)

                def gblk(i, carry):
                    base = (s + i * _NS) * _AU
                    pltpu.sync_copy(pdst.at[r, q, pl.ds(base, _AU)], idxd)
                    _remap(idxd, idxd2, lo, rng, _AU)
                    pltpu.sync_copy(ones_v.at[pl.ds(0, _B)],
                                    acc.at[idxd2.at[pl.ds(0, _B)]], add=True)
                    pltpu.sync_copy(ones_v.at[pl.ds(0, _B)],
                                    acc.at[idxd2.at[pl.ds(_B, _B)]], add=True)
                    return carry

                lax.fori_loop(0, n_i, gblk, 0)
            plsc.subcore_barrier()
            pltpu.sync_copy(acc.at[pl.ds(s * rpt, rpt)],
                            out_hbm.at[pl.ds(lo + s * rpt, rpt)])
            plsc.subcore_barrier()

        for half in range(_NC):
            @pl.when(c == half)
            def _():
                for q in range(half, n_ranges, _NC):
                    one_pass(q)

    return pl.kernel(
        body,
        out_type=jax.ShapeDtypeStruct((n_ranges * rng, _D), jnp.float32),
        mesh=mesh,
        compiler_params=pltpu.CompilerParams(needs_layout_passes=False),
        scratch_types=[
            pltpu.VMEM_SHARED((rng + 128, _D), jnp.float32),
            pltpu.VMEM((_NC, _L), jnp.int32),
            pltpu.VMEM((_AU,), jnp.int32),
            pltpu.VMEM((_AU,), jnp.int32),
            pltpu.VMEM((_AU, _D), jnp.float32),
            pltpu.SemaphoreType.DMA,
        ])


_R = 1000


@functools.lru_cache(maxsize=None)
def _combine_kernel(n, n_et):
    def body(*refs):
        aggs = refs[0:n_et]
        recips = refs[n_et:2 * n_et]
        x_ref = refs[2 * n_et]
        wls = refs[2 * n_et + 1:3 * n_et + 1]
        wr_ref = refs[3 * n_et + 1]
        b_ref = refs[3 * n_et + 2]
        out_ref = refs[3 * n_et + 3]
        acc = jnp.dot(x_ref[...], wr_ref[...],
                      preferred_element_type=jnp.float32) + b_ref[...]
        for a, r, w in zip(aggs, recips, wls):
            acc = acc + jnp.dot(a[...] * r[...], w[...],
                                preferred_element_type=jnp.float32)
        out_ref[...] = jnp.maximum(acc, 0.0)

    row_spec = pl.BlockSpec((_R, _D), lambda i: (i, 0))
    one_spec = pl.BlockSpec((_R, 1), lambda i: (i, 0))
    w_spec = pl.BlockSpec((_D, _D), lambda i: (0, 0))
    b_spec = pl.BlockSpec((1, _D), lambda i: (0, 0))
    in_specs = ([row_spec] * n_et + [one_spec] * n_et + [row_spec]
                + [w_spec] * n_et + [w_spec, b_spec])
    return pl.pallas_call(
        body,
        grid=(n // _R,),
        in_specs=in_specs,
        out_specs=row_spec,
        out_shape=jax.ShapeDtypeStruct((n, _D), jnp.float32),
    )


def _pad_blocks(a, nb, fill):
    total = _NS * nb * _B
    a = jnp.concatenate(
        [a, jnp.full((total - a.shape[0],), fill, jnp.int32)])
    return a.reshape(_NS, nb, _B)


def kernel(params, edges):
    parts = {}
    cnt_recip = {}
    for (s, r, d) in _ETS:
        k = _ekey(s, r, d)
        e = edges[k]
        E = e.shape[1]
        n_dst = _NODE[d]
        _, _, rpt = _ranges(n_dst)
        nb = -(-E // (_NS * _B))
        nb += nb % 2
        srcb = _pad_blocks(e[0], nb, 0)
        dstb = _pad_blocks(e[1], nb, n_dst)
        psrc, pdst, gcnt = _part_kernel(n_dst, nb, E)(srcb, dstb)
        G = psrc.shape[2]
        psrc = psrc.reshape(_NC, psrc.shape[1], G * _AU)
        pdst = pdst.reshape(_NC, pdst.shape[1], G * _AU)
        parts[k] = (psrc, pdst, gcnt, E)
        zeros = jnp.zeros((rpt, _D), jnp.float32)
        ones = jnp.ones((_AU, _D), jnp.float32)
        cnt = _cnt_kernel(n_dst, E)(pdst, gcnt, zeros, ones)[:n_dst, 0]
        cnt_recip[k] = (1.0 / jnp.maximum(cnt, 1.0)).reshape(n_dst, 1)

    x = {nt: params["emb"][nt] for nt in _NODE}
    for l in range(2):
        lp = params["l" + str(l)]
        aggs = {}
        for (s, r, d) in _ETS:
            k = _ekey(s, r, d)
            psrc, pdst, gcnt, E = parts[k]
            n_dst = _NODE[d]
            _, _, rpt = _ranges(n_dst)
            zeros = jnp.zeros((rpt, _D), jnp.float32)
            aggs[k] = _agg_kernel(_NODE[s], n_dst, E)(
                x[s], psrc, pdst, gcnt, zeros)
        new_x = {}
        for nt in _NODE:
            ets = [(s, r, d) for (s, r, d) in _ETS if d == nt]
            n_et = len(ets)
            ks = [_ekey(*et) for et in ets]
            wr = sum(lp[k]["W_r"] for k in ks) / n_et
            bb = (sum(lp[k]["b_l"] for k in ks) / n_et).reshape(1, _D)
            args = ([aggs[k] for k in ks]
                    + [cnt_recip[k] for k in ks]
                    + [x[nt]] + [lp[k]["W_l"] / n_et for k in ks]
                    + [wr, bb])
            new_x[nt] = _combine_kernel(_NODE[nt], n_et)(*args)
        x = new_x
    return (x["drug"], x["disease"], x["gene"])


# speedup vs baseline: 1.0063x; 1.0030x over previous
import functools

import jax
import jax.numpy as jnp
from jax import lax
from jax.experimental import pallas as pl
from jax.experimental.pallas import tpu as pltpu
from jax.experimental.pallas import tpu_sc as plsc

_NODE = {"drug": 20000, "disease": 20000, "gene": 60000}
_ETS = [("drug", "targets", "gene"), ("gene", "assoc", "disease"),
        ("gene", "rev_targets", "drug"), ("disease", "rev_assoc", "gene")]
_D = 128
_NC, _NS, _L = 2, 16, 16
_B = 128
_AU = 256


def _ekey(s, r, d):
    return s + "_" + r + "_" + d


def _ranges(n_dst):
    n_ranges = 6 if n_dst > 32000 else 2
    rpt = -(-n_dst // (n_ranges * _NS * 8)) * 8
    return n_ranges, _NS * rpt, rpt


def _remap(dst_ref, out_ref, lo, rng, n=_B):
    for i in range(n // _L):
        v = dst_ref[pl.ds(i * _L, _L)]
        lv = v - lo
        ok = (lv >= 0) & (lv < rng)
        out_ref[pl.ds(i * _L, _L)] = jnp.where(ok, lv, rng + (v & 127))


@functools.lru_cache(maxsize=None)
def _part_kernel(n_dst, nb, E):
    n_ranges, rng, _ = _ranges(n_dst)
    G = -(-E // (2 * _AU)) + 24
    nb2 = nb // _NC
    mesh = plsc.VectorSubcoreMesh(core_axis_name="c", subcore_axis_name="s")

    def body(srcb, dstb, psrc, pdst, counts_out, *scr):
        counters, vsrc, vdst, stage_s2, stage_d2, cvec, srt_s, srt_d = scr
        SW = 2 * _AU
        c = lax.axis_index("c")
        s = lax.axis_index("s")
        iota = jnp.arange(_L, dtype=jnp.int32)

        @pl.when(s == 0)
        def _():
            for q in range(n_ranges):
                counters[q] = 0

        plsc.subcore_barrier()

        def flush_dma(q):
            grp = plsc.fetch_and_add(counters.at[q], 1, subcore_id=0)
            pltpu.sync_copy(stage_s2.at[pl.ds(q * SW, _AU)],
                            psrc.at[c, q, grp])
            pltpu.sync_copy(stage_d2.at[pl.ds(q * SW, _AU)],
                            pdst.at[c, q, grp])

        def blk(j, fills):
            pltpu.sync_copy(srcb.at[s, c * nb2 + j], vsrc)
            pltpu.sync_copy(dstb.at[s, c * nb2 + j], vdst)
            fills = list(fills)
            for i in range(_B // _L):
                sv = vsrc[pl.ds(i * _L, _L)]
                dv = vdst[pl.ds(i * _L, _L)]
                rid = jnp.zeros((_L,), jnp.int32)
                for q in range(1, n_ranges):
                    rid = rid + jnp.where(dv >= q * rng, 1, 0)
                ks, svs = plsc.sort_key_val(rid, sv)
                _, dvs = plsc.sort_key_val(rid, dv)
                srt_s[pl.ds(0, _L)] = svs
                srt_d[pl.ds(0, _L)] = dvs
                off = 0
                for q in range(n_ranges):
                    m = ks == q
                    pc = jnp.sum(jnp.where(m, 1, 0))
                    f = fills[q]
                    stage_s2[pl.ds(q * SW + f, _L)] = srt_s[pl.ds(off, _L)]
                    stage_d2[pl.ds(q * SW + f, _L)] = srt_d[pl.ds(off, _L)]
                    fills[q] = f + pc
                    off = off + pc
            for q in range(n_ranges):
                f = fills[q]

                @pl.when(f >= _AU)
                def _():
                    flush_dma(q)
                    for t in range(_B // _L):
                        stage_s2[pl.ds(q * SW + t * _L, _L)] = \
                            stage_s2[pl.ds(q * SW + _AU + t * _L, _L)]
                        stage_d2[pl.ds(q * SW + t * _L, _L)] = \
                            stage_d2[pl.ds(q * SW + _AU + t * _L, _L)]
                fills[q] = jnp.where(f >= _AU, f - _AU, f)
            return tuple(fills)

        zero = jnp.zeros((), jnp.int32)
        fills = lax.fori_loop(0, nb2, blk, (zero,) * n_ranges)
        pad_s = jnp.zeros((_L,), jnp.int32)
        pad_d = jnp.full((_L,), n_dst, jnp.int32)
        for q in range(n_ranges):
            f = fills[q]

            @pl.when(f > 0)
            def _():
                for t in range(_AU // _L):
                    stage_s2[pl.ds(q * SW + f + t * _L, _L)] = pad_s
                    stage_d2[pl.ds(q * SW + f + t * _L, _L)] = pad_d
                flush_dma(q)

        plsc.subcore_barrier()

        @pl.when(s == 0)
        def _():
            cv = jnp.zeros((_L,), jnp.int32)
            for q in range(n_ranges):
                cv = jnp.where(iota == q, counters[q], cv)
            cvec[...] = cv
            pltpu.sync_copy(cvec, counts_out.at[c])

    return pl.kernel(
        body,
        out_type=(
            jax.ShapeDtypeStruct((_NC, n_ranges, G, _AU), jnp.int32),
            jax.ShapeDtypeStruct((_NC, n_ranges, G, _AU), jnp.int32),
            jax.ShapeDtypeStruct((_NC, _L), jnp.int32),
        ),
        mesh=mesh,
        compiler_params=pltpu.CompilerParams(needs_layout_passes=False),
        scratch_types=(
            [pltpu.SMEM((_L,), jnp.int32),
             pltpu.VMEM((_B,), jnp.int32),
             pltpu.VMEM((_B,), jnp.int32),
             pltpu.VMEM((n_ranges * 2 * _AU,), jnp.int32),
             pltpu.VMEM((n_ranges * 2 * _AU,), jnp.int32),
             pltpu.VMEM((_L,), jnp.int32),
             pltpu.VMEM((2 * _L,), jnp.int32),
             pltpu.VMEM((2 * _L,), jnp.int32)]))


def _group_count(cnts_v, r, q):
    iota = jnp.arange(_L, dtype=jnp.int32)
    return jnp.max(jnp.where(iota == q, cnts_v[r], 0))


@functools.lru_cache(maxsize=None)
def _agg_kernel(n_src, n_dst, E):
    n_ranges, rng, rpt = _ranges(n_dst)
    mesh = plsc.VectorSubcoreMesh(core_axis_name="c", subcore_axis_name="s")

    def body(x_hbm, psrc, pdst, counts_hbm, zeros_hbm, out_hbm,
             acc, cnts_v, idxs, idxd, idxd2, rows, sem, sem2):
        c = lax.axis_index("c")
        s = lax.axis_index("s")
        pltpu.sync_copy(counts_hbm, cnts_v)

        def one_pass(q):
            lo = q * rng
            pltpu.sync_copy(zeros_hbm.at[pl.ds(0, rpt)],
                            acc.at[pl.ds(s * rpt, rpt)])
            plsc.subcore_barrier()
            for r in range(_NC):
                gq = _group_count(cnts_v, r, q)
                n_i = jnp.maximum(0, (gq - s + _NS - 1) // _NS)

                def gblk(i, carry):
                    ci = s + i * _NS
                    pltpu.sync_copy(psrc.at[r, q, ci], idxs)
                    pltpu.sync_copy(pdst.at[r, q, ci], idxd)
                    _remap(idxd, idxd2, lo, rng, _AU)
                    cp0 = pltpu.async_copy(
                        x_hbm.at[idxs.at[pl.ds(0, _B)]],
                        rows.at[pl.ds(0, _B)], sem)
                    cp1 = pltpu.async_copy(
                        x_hbm.at[idxs.at[pl.ds(_B, _B)]],
                        rows.at[pl.ds(_B, _B)], sem2)
                    cp0.wait()
                    pltpu.sync_copy(rows.at[pl.ds(0, _B)],
                                    acc.at[idxd2.at[pl.ds(0, _B)]], add=True)
                    cp1.wait()
                    pltpu.sync_copy(rows.at[pl.ds(_B, _B)],
                                    acc.at[idxd2.at[pl.ds(_B, _B)]], add=True)
                    return carry

                lax.fori_loop(0, n_i, gblk, 0)
            plsc.subcore_barrier()
            pltpu.sync_copy(acc.at[pl.ds(s * rpt, rpt)],
                            out_hbm.at[pl.ds(lo + s * rpt, rpt)])
            plsc.subcore_barrier()

        for half in range(_NC):
            @pl.when(c == half)
            def _():
                for q in range(half, n_ranges, _NC):
                    one_pass(q)

    return pl.kernel(
        body,
        out_type=jax.ShapeDtypeStruct((n_ranges * rng, _D), jnp.float32),
        mesh=mesh,
        compiler_params=pltpu.CompilerParams(needs_layout_passes=False),
        scratch_types=[
            pltpu.VMEM_SHARED((rng + 128, _D), jnp.float32),
            pltpu.VMEM((_NC, _L), jnp.int32),
            pltpu.VMEM((_AU,), jnp.int32),
            pltpu.VMEM((_AU,), jnp.int32),
            pltpu.VMEM((_AU,), jnp.int32),
            pltpu.VMEM((_AU, _D), jnp.float32),
            pltpu.SemaphoreType.DMA,
            pltpu.SemaphoreType.DMA,
        ])


@functools.lru_cache(maxsize=None)
def _cnt_kernel(n_dst, E):
    n_ranges, rng, rpt = _ranges(n_dst)
    mesh = plsc.VectorSubcoreMesh(core_axis_name="c", subcore_axis_name="s")

    def body(pdst, counts_hbm, zeros_hbm, ones_hbm, out_hbm,
             acc, cnts_v, idxd, idxd2, ones_v, sem):
        c = lax.axis_index("c")
        s = lax.axis_index("s")
        pltpu.sync_copy(counts_hbm, cnts_v)
        pltpu.sync_copy(ones_hbm, ones_v)

        def one_pass(q):
            lo = q * rng
            pltpu.sync_copy(zeros_hbm.at[pl.ds(0, rpt)],
                            acc.at[pl.ds(s * rpt, rpt)])
            plsc.subcore_barrier()
            for r in range(_NC):
                gq = _group_count(cnts_v, r, q)
                n_i = jnp.maximum(0, (gq - s + _NS - 1) // _NS)

                def gblk(i, carry):
                    ci = s + i * _NS
                    pltpu.sync_copy(pdst.at[r, q, ci], idxd)
                    _remap(idxd, idxd2, lo, rng, _AU)
                    pltpu.sync_copy(ones_v.at[pl.ds(0, _B)],
                                    acc.at[idxd2.at[pl.ds(0, _B)]], add=True)
                    pltpu.sync_copy(ones_v.at[pl.ds(0, _B)],
                                    acc.at[idxd2.at[pl.ds(_B, _B)]], add=True)
                    return carry

                lax.fori_loop(0, n_i, gblk, 0)
            plsc.subcore_barrier()
            pltpu.sync_copy(acc.at[pl.ds(s * rpt, rpt)],
                            out_hbm.at[pl.ds(lo + s * rpt, rpt)])
            plsc.subcore_barrier()

        for half in range(_NC):
            @pl.when(c == half)
            def _():
                for q in range(half, n_ranges, _NC):
                    one_pass(q)

    return pl.kernel(
        body,
        out_type=jax.ShapeDtypeStruct((n_ranges * rng, _D), jnp.float32),
        mesh=mesh,
        compiler_params=pltpu.CompilerParams(needs_layout_passes=False),
        scratch_types=[
            pltpu.VMEM_SHARED((rng + 128, _D), jnp.float32),
            pltpu.VMEM((_NC, _L), jnp.int32),
            pltpu.VMEM((_AU,), jnp.int32),
            pltpu.VMEM((_AU,), jnp.int32),
            pltpu.VMEM((_AU, _D), jnp.float32),
            pltpu.SemaphoreType.DMA,
        ])


_R = 1000


@functools.lru_cache(maxsize=None)
def _combine_kernel(n, n_et):
    def body(*refs):
        aggs = refs[0:n_et]
        recips = refs[n_et:2 * n_et]
        x_ref = refs[2 * n_et]
        wls = refs[2 * n_et + 1:3 * n_et + 1]
        wr_ref = refs[3 * n_et + 1]
        b_ref = refs[3 * n_et + 2]
        out_ref = refs[3 * n_et + 3]
        acc = jnp.dot(x_ref[...], wr_ref[...],
                      preferred_element_type=jnp.float32) + b_ref[...]
        for a, r, w in zip(aggs, recips, wls):
            acc = acc + jnp.dot(a[...] * r[...], w[...],
                                preferred_element_type=jnp.float32)
        out_ref[...] = jnp.maximum(acc, 0.0)

    row_spec = pl.BlockSpec((_R, _D), lambda i: (i, 0))
    one_spec = pl.BlockSpec((_R, 1), lambda i: (i, 0))
    w_spec = pl.BlockSpec((_D, _D), lambda i: (0, 0))
    b_spec = pl.BlockSpec((1, _D), lambda i: (0, 0))
    in_specs = ([row_spec] * n_et + [one_spec] * n_et + [row_spec]
                + [w_spec] * n_et + [w_spec, b_spec])
    return pl.pallas_call(
        body,
        grid=(n // _R,),
        in_specs=in_specs,
        out_specs=row_spec,
        out_shape=jax.ShapeDtypeStruct((n, _D), jnp.float32),
    )


def _pad_blocks(a, nb, fill):
    total = _NS * nb * _B
    a = jnp.concatenate(
        [a, jnp.full((total - a.shape[0],), fill, jnp.int32)])
    return a.reshape(_NS, nb, _B)


def kernel(params, edges):
    parts = {}
    cnt_recip = {}
    for (s, r, d) in _ETS:
        k = _ekey(s, r, d)
        e = edges[k]
        E = e.shape[1]
        n_dst = _NODE[d]
        _, _, rpt = _ranges(n_dst)
        nb = -(-E // (_NS * _B))
        nb += nb % 2
        srcb = _pad_blocks(e[0], nb, 0)
        dstb = _pad_blocks(e[1], nb, n_dst)
        psrc, pdst, gcnt = _part_kernel(n_dst, nb, E)(srcb, dstb)
        parts[k] = (psrc, pdst, gcnt, E)
        zeros = jnp.zeros((rpt, _D), jnp.float32)
        ones = jnp.ones((_AU, _D), jnp.float32)
        cnt = _cnt_kernel(n_dst, E)(pdst, gcnt, zeros, ones)[:n_dst, 0]
        cnt_recip[k] = (1.0 / jnp.maximum(cnt, 1.0)).reshape(n_dst, 1)

    x = {nt: params["emb"][nt] for nt in _NODE}
    for l in range(2):
        lp = params["l" + str(l)]
        aggs = {}
        for (s, r, d) in _ETS:
            k = _ekey(s, r, d)
            psrc, pdst, gcnt, E = parts[k]
            n_dst = _NODE[d]
            _, _, rpt = _ranges(n_dst)
            zeros = jnp.zeros((rpt, _D), jnp.float32)
            aggs[k] = _agg_kernel(_NODE[s], n_dst, E)(
                x[s], psrc, pdst, gcnt, zeros)
        new_x = {}
        for nt in _NODE:
            ets = [(s, r, d) for (s, r, d) in _ETS if d == nt]
            n_et = len(ets)
            ks = [_ekey(*et) for et in ets]
            wr = sum(lp[k]["W_r"] for k in ks) / n_et
            bb = (sum(lp[k]["b_l"] for k in ks) / n_et).reshape(1, _D)
            args = ([aggs[k] for k in ks]
                    + [cnt_recip[k] for k in ks]
                    + [x[nt]] + [lp[k]["W_l"] / n_et for k in ks]
                    + [wr, bb])
            new_x[nt] = _combine_kernel(_NODE[nt], n_et)(*args)
        x = new_x
    return (x["drug"], x["disease"], x["gene"])


# speedup vs baseline: 1.7496x; 1.7386x over previous
import functools

import jax
import jax.numpy as jnp
from jax import lax
from jax.experimental import pallas as pl
from jax.experimental.pallas import tpu as pltpu
from jax.experimental.pallas import tpu_sc as plsc

_NODE = {"drug": 20000, "disease": 20000, "gene": 60000}
_ETS = [("drug", "targets", "gene"), ("gene", "assoc", "disease"),
        ("gene", "rev_targets", "drug"), ("disease", "rev_assoc", "gene")]
_D = 128
_NC, _NS, _L = 2, 16, 16
_B = 128
_AU = 128


def _ekey(s, r, d):
    return s + "_" + r + "_" + d


def _ranges(n_dst):
    n_ranges = 6 if n_dst > 32000 else 2
    rpt = -(-n_dst // (n_ranges * _NS * 8)) * 8
    return n_ranges, _NS * rpt, rpt


def _remap(dst_ref, out_ref, lo, rng, n=_B):
    for i in range(n // _L):
        v = dst_ref[pl.ds(i * _L, _L)]
        lv = v - lo
        ok = (lv >= 0) & (lv < rng)
        out_ref[pl.ds(i * _L, _L)] = jnp.where(ok, lv, rng + (v & 127))


@functools.lru_cache(maxsize=None)
def _part_kernel(n_dst, nb, E):
    n_ranges, rng, _ = _ranges(n_dst)
    G = -(-E // (2 * _AU)) + 24
    nb2 = nb // _NC
    mesh = plsc.VectorSubcoreMesh(core_axis_name="c", subcore_axis_name="s")

    def body(srcb, dstb, psrc, pdst, counts_out, *scr):
        counters, vsrc, vdst, stage_s2, stage_d2, cvec, srt_s, srt_d = scr
        SW = 2 * _AU
        c = lax.axis_index("c")
        s = lax.axis_index("s")
        iota = jnp.arange(_L, dtype=jnp.int32)

        @pl.when(s == 0)
        def _():
            for q in range(n_ranges):
                counters[q] = 0

        plsc.subcore_barrier()

        def flush_dma(q):
            grp = plsc.fetch_and_add(counters.at[q], 1, subcore_id=0)
            pltpu.sync_copy(stage_s2.at[pl.ds(q * SW, _AU)],
                            psrc.at[c, q, grp])
            pltpu.sync_copy(stage_d2.at[pl.ds(q * SW, _AU)],
                            pdst.at[c, q, grp])

        def blk(j, fills):
            pltpu.sync_copy(srcb.at[s, c * nb2 + j], vsrc)
            pltpu.sync_copy(dstb.at[s, c * nb2 + j], vdst)
            fills = list(fills)
            for i in range(_B // _L):
                sv = vsrc[pl.ds(i * _L, _L)]
                dv = vdst[pl.ds(i * _L, _L)]
                rid = jnp.zeros((_L,), jnp.int32)
                for q in range(1, n_ranges):
                    rid = rid + jnp.where(dv >= q * rng, 1, 0)
                ks, svs = plsc.sort_key_val(rid, sv)
                _, dvs = plsc.sort_key_val(rid, dv)
                srt_s[pl.ds(0, _L)] = svs
                srt_d[pl.ds(0, _L)] = dvs
                off = 0
                for q in range(n_ranges):
                    m = ks == q
                    pc = jnp.sum(jnp.where(m, 1, 0))
                    f = fills[q]
                    stage_s2[pl.ds(q * SW + f, _L)] = srt_s[pl.ds(off, _L)]
                    stage_d2[pl.ds(q * SW + f, _L)] = srt_d[pl.ds(off, _L)]
                    fills[q] = f + pc
                    off = off + pc
            for q in range(n_ranges):
                f = fills[q]

                @pl.when(f >= _AU)
                def _():
                    flush_dma(q)
                    for t in range(_B // _L):
                        stage_s2[pl.ds(q * SW + t * _L, _L)] = \
                            stage_s2[pl.ds(q * SW + _AU + t * _L, _L)]
                        stage_d2[pl.ds(q * SW + t * _L, _L)] = \
                            stage_d2[pl.ds(q * SW + _AU + t * _L, _L)]
                fills[q] = jnp.where(f >= _AU, f - _AU, f)
            return tuple(fills)

        zero = jnp.zeros((), jnp.int32)
        fills = lax.fori_loop(0, nb2, blk, (zero,) * n_ranges)
        pad_s = jnp.zeros((_L,), jnp.int32)
        pad_d = jnp.full((_L,), n_dst, jnp.int32)
        for q in range(n_ranges):
            f = fills[q]

            @pl.when(f > 0)
            def _():
                for t in range(_AU // _L):
                    stage_s2[pl.ds(q * SW + f + t * _L, _L)] = pad_s
                    stage_d2[pl.ds(q * SW + f + t * _L, _L)] = pad_d
                flush_dma(q)

        plsc.subcore_barrier()

        @pl.when(s == 0)
        def _():
            cv = jnp.zeros((_L,), jnp.int32)
            for q in range(n_ranges):
                cv = jnp.where(iota == q, counters[q], cv)
            cvec[...] = cv
            pltpu.sync_copy(cvec, counts_out.at[c])

    return pl.kernel(
        body,
        out_type=(
            jax.ShapeDtypeStruct((_NC, n_ranges, G, _AU), jnp.int32),
            jax.ShapeDtypeStruct((_NC, n_ranges, G, _AU), jnp.int32),
            jax.ShapeDtypeStruct((_NC, _L), jnp.int32),
        ),
        mesh=mesh,
        compiler_params=pltpu.CompilerParams(needs_layout_passes=False),
        scratch_types=(
            [pltpu.SMEM((_L,), jnp.int32),
             pltpu.VMEM((_B,), jnp.int32),
             pltpu.VMEM((_B,), jnp.int32),
             pltpu.VMEM((n_ranges * 2 * _AU,), jnp.int32),
             pltpu.VMEM((n_ranges * 2 * _AU,), jnp.int32),
             pltpu.VMEM((_L,), jnp.int32),
             pltpu.VMEM((2 * _L,), jnp.int32),
             pltpu.VMEM((2 * _L,), jnp.int32)]))


def _group_count(cnts_v, r, q):
    iota = jnp.arange(_L, dtype=jnp.int32)
    return jnp.max(jnp.where(iota == q, cnts_v[r], 0))


@functools.lru_cache(maxsize=None)
def _agg_kernel(n_src, n_dst, E):
    n_ranges, rng, rpt = _ranges(n_dst)
    mesh = plsc.VectorSubcoreMesh(core_axis_name="c", subcore_axis_name="s")

    def body(x_hbm, psrc, pdst, counts_hbm, zeros_hbm, out_hbm,
             acc, cnts_v, idxs, idxd, idxd2, rows, sem, sem2):
        c = lax.axis_index("c")
        s = lax.axis_index("s")
        pltpu.sync_copy(counts_hbm, cnts_v)

        def one_pass(q):
            lo = q * rng
            pltpu.sync_copy(zeros_hbm.at[pl.ds(0, rpt)],
                            acc.at[pl.ds(s * rpt, rpt)])
            plsc.subcore_barrier()
            for r in range(_NC):
                gq = _group_count(cnts_v, r, q)
                n_i = jnp.maximum(0, (gq - s + _NS - 1) // _NS)

                def gblk(i, carry):
                    ci = s + i * _NS
                    pltpu.sync_copy(psrc.at[r, q, ci], idxs)
                    pltpu.sync_copy(pdst.at[r, q, ci], idxd)
                    _remap(idxd, idxd2, lo, rng, _AU)
                    pltpu.async_copy(x_hbm.at[idxs], rows, sem).wait()
                    pltpu.sync_copy(rows, acc.at[idxd2], add=True)
                    return carry

                lax.fori_loop(0, n_i, gblk, 0)
            plsc.subcore_barrier()
            pltpu.sync_copy(acc.at[pl.ds(s * rpt, rpt)],
                            out_hbm.at[pl.ds(lo + s * rpt, rpt)])
            plsc.subcore_barrier()

        for half in range(_NC):
            @pl.when(c == half)
            def _():
                for q in range(half, n_ranges, _NC):
                    one_pass(q)

    return pl.kernel(
        body,
        out_type=jax.ShapeDtypeStruct((n_ranges * rng, _D), jnp.float32),
        mesh=mesh,
        compiler_params=pltpu.CompilerParams(needs_layout_passes=False),
        scratch_types=[
            pltpu.VMEM_SHARED((rng + 128, _D), jnp.float32),
            pltpu.VMEM((_NC, _L), jnp.int32),
            pltpu.VMEM((_AU,), jnp.int32),
            pltpu.VMEM((_AU,), jnp.int32),
            pltpu.VMEM((_AU,), jnp.int32),
            pltpu.VMEM((_AU, _D), jnp.float32),
            pltpu.SemaphoreType.DMA,
            pltpu.SemaphoreType.DMA,
        ])


@functools.lru_cache(maxsize=None)
def _cnt_kernel(n_dst, E):
    n_ranges, rng, rpt = _ranges(n_dst)
    mesh = plsc.VectorSubcoreMesh(core_axis_name="c", subcore_axis_name="s")

    def body(pdst, counts_hbm, zeros_hbm, ones_hbm, out_hbm,
             acc, cnts_v, idxd, idxd2, ones_v, sem):
        c = lax.axis_index("c")
        s = lax.axis_index("s")
        pltpu.sync_copy(counts_hbm, cnts_v)
        pltpu.sync_copy(ones_hbm, ones_v)

        def one_pass(q):
            lo = q * rng
            pltpu.sync_copy(zeros_hbm.at[pl.ds(0, rpt)],
                            acc.at[pl.ds(s * rpt, rpt)])
            plsc.subcore_barrier()
            for r in range(_NC):
                gq = _group_count(cnts_v, r, q)
                n_i = jnp.maximum(0, (gq - s + _NS - 1) // _NS)

                def gblk(i, carry):
                    ci = s + i * _NS
                    pltpu.sync_copy(pdst.at[r, q, ci], idxd)
                    _remap(idxd, idxd2, lo, rng, _AU)
                    pltpu.sync_copy(ones_v, acc.at[idxd2], add=True)
                    return carry

                lax.fori_loop(0, n_i, gblk, 0)
            plsc.subcore_barrier()
            pltpu.sync_copy(acc.at[pl.ds(s * rpt, rpt)],
                            out_hbm.at[pl.ds(lo + s * rpt, rpt)])
            plsc.subcore_barrier()

        for half in range(_NC):
            @pl.when(c == half)
            def _():
                for q in range(half, n_ranges, _NC):
                    one_pass(q)

    return pl.kernel(
        body,
        out_type=jax.ShapeDtypeStruct((n_ranges * rng, _D), jnp.float32),
        mesh=mesh,
        compiler_params=pltpu.CompilerParams(needs_layout_passes=False),
        scratch_types=[
            pltpu.VMEM_SHARED((rng + 128, _D), jnp.float32),
            pltpu.VMEM((_NC, _L), jnp.int32),
            pltpu.VMEM((_AU,), jnp.int32),
            pltpu.VMEM((_AU,), jnp.int32),
            pltpu.VMEM((_AU, _D), jnp.float32),
            pltpu.SemaphoreType.DMA,
        ])


_R = 1000


@functools.lru_cache(maxsize=None)
def _combine_kernel(n, n_et):
    def body(*refs):
        aggs = refs[0:n_et]
        recips = refs[n_et:2 * n_et]
        x_ref = refs[2 * n_et]
        wls = refs[2 * n_et + 1:3 * n_et + 1]
        wr_ref = refs[3 * n_et + 1]
        b_ref = refs[3 * n_et + 2]
        out_ref = refs[3 * n_et + 3]
        acc = jnp.dot(x_ref[...], wr_ref[...],
                      preferred_element_type=jnp.float32) + b_ref[...]
        for a, r, w in zip(aggs, recips, wls):
            acc = acc + jnp.dot(a[...] * r[...], w[...],
                                preferred_element_type=jnp.float32)
        out_ref[...] = jnp.maximum(acc, 0.0)

    row_spec = pl.BlockSpec((_R, _D), lambda i: (i, 0))
    one_spec = pl.BlockSpec((_R, 1), lambda i: (i, 0))
    w_spec = pl.BlockSpec((_D, _D), lambda i: (0, 0))
    b_spec = pl.BlockSpec((1, _D), lambda i: (0, 0))
    in_specs = ([row_spec] * n_et + [one_spec] * n_et + [row_spec]
                + [w_spec] * n_et + [w_spec, b_spec])
    return pl.pallas_call(
        body,
        grid=(n // _R,),
        in_specs=in_specs,
        out_specs=row_spec,
        out_shape=jax.ShapeDtypeStruct((n, _D), jnp.float32),
    )


def _pad_blocks(a, nb, fill):
    total = _NS * nb * _B
    a = jnp.concatenate(
        [a, jnp.full((total - a.shape[0],), fill, jnp.int32)])
    return a.reshape(_NS, nb, _B)


def kernel(params, edges):
    parts = {}
    cnt_recip = {}
    for (s, r, d) in _ETS:
        k = _ekey(s, r, d)
        e = edges[k]
        E = e.shape[1]
        n_dst = _NODE[d]
        _, _, rpt = _ranges(n_dst)
        nb = -(-E // (_NS * _B))
        nb += nb % 2
        srcb = _pad_blocks(e[0], nb, 0)
        dstb = _pad_blocks(e[1], nb, n_dst)
        psrc, pdst, gcnt = _part_kernel(n_dst, nb, E)(srcb, dstb)
        parts[k] = (psrc, pdst, gcnt, E)
        zeros = jnp.zeros((rpt, _D), jnp.float32)
        ones = jnp.ones((_AU, _D), jnp.float32)
        cnt = _cnt_kernel(n_dst, E)(pdst, gcnt, zeros, ones)[:n_dst, 0]
        cnt_recip[k] = (1.0 / jnp.maximum(cnt, 1.0)).reshape(n_dst, 1)

    x = {nt: params["emb"][nt] for nt in _NODE}
    for l in range(2):
        lp = params["l" + str(l)]
        aggs = {}
        for (s, r, d) in _ETS:
            k = _ekey(s, r, d)
            psrc, pdst, gcnt, E = parts[k]
            n_dst = _NODE[d]
            _, _, rpt = _ranges(n_dst)
            zeros = jnp.zeros((rpt, _D), jnp.float32)
            aggs[k] = _agg_kernel(_NODE[s], n_dst, E)(
                x[s], psrc, pdst, gcnt, zeros)
        new_x = {}
        for nt in _NODE:
            ets = [(s, r, d) for (s, r, d) in _ETS if d == nt]
            n_et = len(ets)
            ks = [_ekey(*et) for et in ets]
            wr = sum(lp[k]["W_r"] for k in ks) / n_et
            bb = (sum(lp[k]["b_l"] for k in ks) / n_et).reshape(1, _D)
            args = ([aggs[k] for k in ks]
                    + [cnt_recip[k] for k in ks]
                    + [x[nt]] + [lp[k]["W_l"] / n_et for k in ks]
                    + [wr, bb])
            new_x[nt] = _combine_kernel(_NODE[nt], n_et)(*args)
        x = new_x
    return (x["drug"], x["disease"], x["gene"])


# speedup vs baseline: 1.7787x; 1.0166x over previous
import functools

import jax
import jax.numpy as jnp
from jax import lax
from jax.experimental import pallas as pl
from jax.experimental.pallas import tpu as pltpu
from jax.experimental.pallas import tpu_sc as plsc

_NODE = {"drug": 20000, "disease": 20000, "gene": 60000}
_ETS = [("drug", "targets", "gene"), ("gene", "assoc", "disease"),
        ("gene", "rev_targets", "drug"), ("disease", "rev_assoc", "gene")]
_D = 128
_NC, _NS, _L = 2, 16, 16
_B = 128
_AU = 128


def _ekey(s, r, d):
    return s + "_" + r + "_" + d


def _ranges(n_dst):
    n_ranges = 6 if n_dst > 32000 else 2
    rpt = -(-n_dst // (n_ranges * _NS * 8)) * 8
    return n_ranges, _NS * rpt, rpt


def _remap(dst_ref, out_ref, lo, rng, n=_B):
    for i in range(n // _L):
        v = dst_ref[pl.ds(i * _L, _L)]
        lv = v - lo
        ok = (lv >= 0) & (lv < rng)
        out_ref[pl.ds(i * _L, _L)] = jnp.where(ok, lv, rng + (v & 127))


@functools.lru_cache(maxsize=None)
def _part_kernel(n_dst, nb, E):
    n_ranges, rng, _ = _ranges(n_dst)
    G = -(-E // (2 * _AU)) + 24
    nb2 = nb // _NC
    mesh = plsc.VectorSubcoreMesh(core_axis_name="c", subcore_axis_name="s")

    def body(srcb, dstb, psrc, pdst, counts_out, *scr):
        counters, vsrc, vdst, stage_s2, stage_d2, cvec, srt_s, srt_d = scr
        SW = 2 * _AU
        c = lax.axis_index("c")
        s = lax.axis_index("s")
        iota = jnp.arange(_L, dtype=jnp.int32)

        @pl.when(s == 0)
        def _():
            for q in range(n_ranges):
                counters[q] = 0

        plsc.subcore_barrier()

        def flush_dma(q):
            grp = plsc.fetch_and_add(counters.at[q], 1, subcore_id=0)
            pltpu.sync_copy(stage_s2.at[pl.ds(q * SW, _AU)],
                            psrc.at[c, q, grp])
            pltpu.sync_copy(stage_d2.at[pl.ds(q * SW, _AU)],
                            pdst.at[c, q, grp])

        def blk(j, fills):
            pltpu.sync_copy(srcb.at[s, c * nb2 + j], vsrc)
            pltpu.sync_copy(dstb.at[s, c * nb2 + j], vdst)
            fills = list(fills)
            for i in range(_B // _L):
                sv = vsrc[pl.ds(i * _L, _L)]
                dv = vdst[pl.ds(i * _L, _L)]
                rid = jnp.zeros((_L,), jnp.int32)
                for q in range(1, n_ranges):
                    rid = rid + jnp.where(dv >= q * rng, 1, 0)
                ks, svs = plsc.sort_key_val(rid, sv)
                _, dvs = plsc.sort_key_val(rid, dv)
                srt_s[pl.ds(0, _L)] = svs
                srt_d[pl.ds(0, _L)] = dvs
                off = 0
                for q in range(n_ranges):
                    m = ks == q
                    pc = jnp.sum(jnp.where(m, 1, 0))
                    f = fills[q]
                    stage_s2[pl.ds(q * SW + f, _L)] = srt_s[pl.ds(off, _L)]
                    stage_d2[pl.ds(q * SW + f, _L)] = srt_d[pl.ds(off, _L)]
                    fills[q] = f + pc
                    off = off + pc
            for q in range(n_ranges):
                f = fills[q]

                @pl.when(f >= _AU)
                def _():
                    flush_dma(q)
                    for t in range(_B // _L):
                        stage_s2[pl.ds(q * SW + t * _L, _L)] = \
                            stage_s2[pl.ds(q * SW + _AU + t * _L, _L)]
                        stage_d2[pl.ds(q * SW + t * _L, _L)] = \
                            stage_d2[pl.ds(q * SW + _AU + t * _L, _L)]
                fills[q] = jnp.where(f >= _AU, f - _AU, f)
            return tuple(fills)

        zero = jnp.zeros((), jnp.int32)
        fills = lax.fori_loop(0, nb2, blk, (zero,) * n_ranges)
        pad_s = jnp.zeros((_L,), jnp.int32)
        pad_d = jnp.full((_L,), n_dst, jnp.int32)
        for q in range(n_ranges):
            f = fills[q]

            @pl.when(f > 0)
            def _():
                for t in range(_AU // _L):
                    stage_s2[pl.ds(q * SW + f + t * _L, _L)] = pad_s
                    stage_d2[pl.ds(q * SW + f + t * _L, _L)] = pad_d
                flush_dma(q)

        plsc.subcore_barrier()

        @pl.when(s == 0)
        def _():
            cv = jnp.zeros((_L,), jnp.int32)
            for q in range(n_ranges):
                cv = jnp.where(iota == q, counters[q], cv)
            cvec[...] = cv
            pltpu.sync_copy(cvec, counts_out.at[c])

    return pl.kernel(
        body,
        out_type=(
            jax.ShapeDtypeStruct((_NC, n_ranges, G, _AU), jnp.int32),
            jax.ShapeDtypeStruct((_NC, n_ranges, G, _AU), jnp.int32),
            jax.ShapeDtypeStruct((_NC, _L), jnp.int32),
        ),
        mesh=mesh,
        compiler_params=pltpu.CompilerParams(needs_layout_passes=False),
        scratch_types=(
            [pltpu.SMEM((_L,), jnp.int32),
             pltpu.VMEM((_B,), jnp.int32),
             pltpu.VMEM((_B,), jnp.int32),
             pltpu.VMEM((n_ranges * 2 * _AU,), jnp.int32),
             pltpu.VMEM((n_ranges * 2 * _AU,), jnp.int32),
             pltpu.VMEM((_L,), jnp.int32),
             pltpu.VMEM((2 * _L,), jnp.int32),
             pltpu.VMEM((2 * _L,), jnp.int32)]))


def _group_count(cnts_v, r, q):
    iota = jnp.arange(_L, dtype=jnp.int32)
    return jnp.max(jnp.where(iota == q, cnts_v[r], 0))


@functools.lru_cache(maxsize=None)
def _agg_kernel(n_src, n_dst, E):
    n_ranges, rng, rpt = _ranges(n_dst)
    mesh = plsc.VectorSubcoreMesh(core_axis_name="c", subcore_axis_name="s")

    def body(x_hbm, psrc, pdst, counts_hbm, zeros_hbm, out_hbm,
             acc, cnts_v, idxs, idxd, idxd2, rows, sem, sem2):
        c = lax.axis_index("c")
        s = lax.axis_index("s")
        pltpu.sync_copy(counts_hbm, cnts_v)

        def one_pass(q):
            lo = q * rng
            pltpu.sync_copy(zeros_hbm.at[pl.ds(0, rpt)],
                            acc.at[pl.ds(s * rpt, rpt)])
            plsc.subcore_barrier()
            for r in range(_NC):
                gq = _group_count(cnts_v, r, q)
                n_i = jnp.maximum(0, (gq - s + _NS - 1) // _NS)

                def gblk(i, carry):
                    ci = s + i * _NS
                    pltpu.sync_copy(psrc.at[r, q, ci], idxs)
                    cp = pltpu.async_copy(x_hbm.at[idxs], rows, sem)
                    pltpu.sync_copy(pdst.at[r, q, ci], idxd)
                    _remap(idxd, idxd2, lo, rng, _AU)
                    cp.wait()
                    pltpu.sync_copy(rows, acc.at[idxd2], add=True)
                    return carry

                lax.fori_loop(0, n_i, gblk, 0)
            plsc.subcore_barrier()
            pltpu.sync_copy(acc.at[pl.ds(s * rpt, rpt)],
                            out_hbm.at[pl.ds(lo + s * rpt, rpt)])
            plsc.subcore_barrier()

        for half in range(_NC):
            @pl.when(c == half)
            def _():
                for q in range(half, n_ranges, _NC):
                    one_pass(q)

    return pl.kernel(
        body,
        out_type=jax.ShapeDtypeStruct((n_ranges * rng, _D), jnp.float32),
        mesh=mesh,
        compiler_params=pltpu.CompilerParams(needs_layout_passes=False),
        scratch_types=[
            pltpu.VMEM_SHARED((rng + 128, _D), jnp.float32),
            pltpu.VMEM((_NC, _L), jnp.int32),
            pltpu.VMEM((_AU,), jnp.int32),
            pltpu.VMEM((_AU,), jnp.int32),
            pltpu.VMEM((_AU,), jnp.int32),
            pltpu.VMEM((_AU, _D), jnp.float32),
            pltpu.SemaphoreType.DMA,
            pltpu.SemaphoreType.DMA,
        ])


@functools.lru_cache(maxsize=None)
def _cnt_kernel(n_dst, E):
    n_ranges, rng, rpt = _ranges(n_dst)
    mesh = plsc.VectorSubcoreMesh(core_axis_name="c", subcore_axis_name="s")

    def body(pdst, counts_hbm, zeros_hbm, ones_hbm, out_hbm,
             acc, cnts_v, idxd, idxd2, ones_v, sem):
        c = lax.axis_index("c")
        s = lax.axis_index("s")
        pltpu.sync_copy(counts_hbm, cnts_v)
        pltpu.sync_copy(ones_hbm, ones_v)

        def one_pass(q):
            lo = q * rng
            pltpu.sync_copy(zeros_hbm.at[pl.ds(0, rpt)],
                            acc.at[pl.ds(s * rpt, rpt)])
            plsc.subcore_barrier()
            for r in range(_NC):
                gq = _group_count(cnts_v, r, q)
                n_i = jnp.maximum(0, (gq - s + _NS - 1) // _NS)

                def gblk(i, carry):
                    ci = s + i * _NS
                    pltpu.sync_copy(pdst.at[r, q, ci], idxd)
                    _remap(idxd, idxd2, lo, rng, _AU)
                    pltpu.sync_copy(ones_v, acc.at[idxd2], add=True)
                    return carry

                lax.fori_loop(0, n_i, gblk, 0)
            plsc.subcore_barrier()
            pltpu.sync_copy(acc.at[pl.ds(s * rpt, rpt)],
                            out_hbm.at[pl.ds(lo + s * rpt, rpt)])
            plsc.subcore_barrier()

        for half in range(_NC):
            @pl.when(c == half)
            def _():
                for q in range(half, n_ranges, _NC):
                    one_pass(q)

    return pl.kernel(
        body,
        out_type=jax.ShapeDtypeStruct((n_ranges * rng, _D), jnp.float32),
        mesh=mesh,
        compiler_params=pltpu.CompilerParams(needs_layout_passes=False),
        scratch_types=[
            pltpu.VMEM_SHARED((rng + 128, _D), jnp.float32),
            pltpu.VMEM((_NC, _L), jnp.int32),
            pltpu.VMEM((_AU,), jnp.int32),
            pltpu.VMEM((_AU,), jnp.int32),
            pltpu.VMEM((_AU, _D), jnp.float32),
            pltpu.SemaphoreType.DMA,
        ])


_R = 1000


@functools.lru_cache(maxsize=None)
def _combine_kernel(n, n_et):
    def body(*refs):
        aggs = refs[0:n_et]
        recips = refs[n_et:2 * n_et]
        x_ref = refs[2 * n_et]
        wls = refs[2 * n_et + 1:3 * n_et + 1]
        wr_ref = refs[3 * n_et + 1]
        b_ref = refs[3 * n_et + 2]
        out_ref = refs[3 * n_et + 3]
        acc = jnp.dot(x_ref[...], wr_ref[...],
                      preferred_element_type=jnp.float32) + b_ref[...]
        for a, r, w in zip(aggs, recips, wls):
            acc = acc + jnp.dot(a[...] * r[...], w[...],
                                preferred_element_type=jnp.float32)
        out_ref[...] = jnp.maximum(acc, 0.0)

    row_spec = pl.BlockSpec((_R, _D), lambda i: (i, 0))
    one_spec = pl.BlockSpec((_R, 1), lambda i: (i, 0))
    w_spec = pl.BlockSpec((_D, _D), lambda i: (0, 0))
    b_spec = pl.BlockSpec((1, _D), lambda i: (0, 0))
    in_specs = ([row_spec] * n_et + [one_spec] * n_et + [row_spec]
                + [w_spec] * n_et + [w_spec, b_spec])
    return pl.pallas_call(
        body,
        grid=(n // _R,),
        in_specs=in_specs,
        out_specs=row_spec,
        out_shape=jax.ShapeDtypeStruct((n, _D), jnp.float32),
    )


def _pad_blocks(a, nb, fill):
    total = _NS * nb * _B
    a = jnp.concatenate(
        [a, jnp.full((total - a.shape[0],), fill, jnp.int32)])
    return a.reshape(_NS, nb, _B)


def kernel(params, edges):
    parts = {}
    cnt_recip = {}
    for (s, r, d) in _ETS:
        k = _ekey(s, r, d)
        e = edges[k]
        E = e.shape[1]
        n_dst = _NODE[d]
        _, _, rpt = _ranges(n_dst)
        nb = -(-E // (_NS * _B))
        nb += nb % 2
        srcb = _pad_blocks(e[0], nb, 0)
        dstb = _pad_blocks(e[1], nb, n_dst)
        psrc, pdst, gcnt = _part_kernel(n_dst, nb, E)(srcb, dstb)
        parts[k] = (psrc, pdst, gcnt, E)
        zeros = jnp.zeros((rpt, _D), jnp.float32)
        ones = jnp.ones((_AU, _D), jnp.float32)
        cnt = _cnt_kernel(n_dst, E)(pdst, gcnt, zeros, ones)[:n_dst, 0]
        cnt_recip[k] = (1.0 / jnp.maximum(cnt, 1.0)).reshape(n_dst, 1)

    x = {nt: params["emb"][nt] for nt in _NODE}
    for l in range(2):
        lp = params["l" + str(l)]
        aggs = {}
        for (s, r, d) in _ETS:
            k = _ekey(s, r, d)
            psrc, pdst, gcnt, E = parts[k]
            n_dst = _NODE[d]
            _, _, rpt = _ranges(n_dst)
            zeros = jnp.zeros((rpt, _D), jnp.float32)
            aggs[k] = _agg_kernel(_NODE[s], n_dst, E)(
                x[s], psrc, pdst, gcnt, zeros)
        new_x = {}
        for nt in _NODE:
            ets = [(s, r, d) for (s, r, d) in _ETS if d == nt]
            n_et = len(ets)
            ks = [_ekey(*et) for et in ets]
            wr = sum(lp[k]["W_r"] for k in ks) / n_et
            bb = (sum(lp[k]["b_l"] for k in ks) / n_et).reshape(1, _D)
            args = ([aggs[k] for k in ks]
                    + [cnt_recip[k] for k in ks]
                    + [x[nt]] + [lp[k]["W_l"] / n_et for k in ks]
                    + [wr, bb])
            new_x[nt] = _combine_kernel(_NODE[nt], n_et)(*args)
        x = new_x
    return (x["drug"], x["disease"], x["gene"])


# speedup vs baseline: 1.7810x; 1.0013x over previous
import functools

import jax
import jax.numpy as jnp
from jax import lax
from jax.experimental import pallas as pl
from jax.experimental.pallas import tpu as pltpu
from jax.experimental.pallas import tpu_sc as plsc

_NODE = {"drug": 20000, "disease": 20000, "gene": 60000}
_ETS = [("drug", "targets", "gene"), ("gene", "assoc", "disease"),
        ("gene", "rev_targets", "drug"), ("disease", "rev_assoc", "gene")]
_D = 128
_NC, _NS, _L = 2, 16, 16
_B = 128
_AU = 128


def _ekey(s, r, d):
    return s + "_" + r + "_" + d


def _ranges(n_dst):
    n_ranges = 6 if n_dst > 32000 else 2
    rpt = -(-n_dst // (n_ranges * _NS * 8)) * 8
    return n_ranges, _NS * rpt, rpt


def _remap(dst_ref, out_ref, lo, rng, n=_B):
    for i in range(n // _L):
        v = dst_ref[pl.ds(i * _L, _L)]
        lv = v - lo
        ok = (lv >= 0) & (lv < rng)
        out_ref[pl.ds(i * _L, _L)] = jnp.where(ok, lv, rng + (v & 127))


@functools.lru_cache(maxsize=None)
def _part_kernel(n_dst, nb, E):
    n_ranges, rng, _ = _ranges(n_dst)
    G = -(-E // (2 * _AU)) + 24
    nb2 = nb // _NC
    mesh = plsc.VectorSubcoreMesh(core_axis_name="c", subcore_axis_name="s")

    def body(srcb, dstb, psrc, pdst, counts_out, *scr):
        counters, vsrc, vdst, stage_s2, stage_d2, cvec, srt_s, srt_d = scr
        SW = 2 * _AU
        c = lax.axis_index("c")
        s = lax.axis_index("s")
        iota = jnp.arange(_L, dtype=jnp.int32)

        @pl.when(s == 0)
        def _():
            for q in range(n_ranges):
                counters[q] = 0

        plsc.subcore_barrier()

        def flush_dma(q):
            grp = plsc.fetch_and_add(counters.at[q], 1, subcore_id=0)
            pltpu.sync_copy(stage_s2.at[pl.ds(q * SW, _AU)],
                            psrc.at[c, q, grp])
            pltpu.sync_copy(stage_d2.at[pl.ds(q * SW, _AU)],
                            pdst.at[c, q, grp])

        def blk(j, fills):
            pltpu.sync_copy(srcb.at[s, c * nb2 + j], vsrc)
            pltpu.sync_copy(dstb.at[s, c * nb2 + j], vdst)
            fills = list(fills)
            for i in range(_B // _L):
                sv = vsrc[pl.ds(i * _L, _L)]
                dv = vdst[pl.ds(i * _L, _L)]
                rid = jnp.zeros((_L,), jnp.int32)
                for q in range(1, n_ranges):
                    rid = rid + jnp.where(dv >= q * rng, 1, 0)
                ks, svs = plsc.sort_key_val(rid, sv)
                _, dvs = plsc.sort_key_val(rid, dv)
                srt_s[pl.ds(0, _L)] = svs
                srt_d[pl.ds(0, _L)] = dvs
                off = 0
                for q in range(n_ranges):
                    m = ks == q
                    pc = jnp.sum(jnp.where(m, 1, 0))
                    f = fills[q]
                    stage_s2[pl.ds(q * SW + f, _L)] = srt_s[pl.ds(off, _L)]
                    stage_d2[pl.ds(q * SW + f, _L)] = srt_d[pl.ds(off, _L)]
                    fills[q] = f + pc
                    off = off + pc
            for q in range(n_ranges):
                f = fills[q]

                @pl.when(f >= _AU)
                def _():
                    flush_dma(q)
                    for t in range(_B // _L):
                        stage_s2[pl.ds(q * SW + t * _L, _L)] = \
                            stage_s2[pl.ds(q * SW + _AU + t * _L, _L)]
                        stage_d2[pl.ds(q * SW + t * _L, _L)] = \
                            stage_d2[pl.ds(q * SW + _AU + t * _L, _L)]
                fills[q] = jnp.where(f >= _AU, f - _AU, f)
            return tuple(fills)

        zero = jnp.zeros((), jnp.int32)
        fills = lax.fori_loop(0, nb2, blk, (zero,) * n_ranges)
        pad_s = jnp.zeros((_L,), jnp.int32)
        pad_d = jnp.full((_L,), n_dst, jnp.int32)
        for q in range(n_ranges):
            f = fills[q]

            @pl.when(f > 0)
            def _():
                for t in range(_AU // _L):
                    stage_s2[pl.ds(q * SW + f + t * _L, _L)] = pad_s
                    stage_d2[pl.ds(q * SW + f + t * _L, _L)] = pad_d
                flush_dma(q)

        plsc.subcore_barrier()

        @pl.when(s == 0)
        def _():
            cv = jnp.zeros((_L,), jnp.int32)
            for q in range(n_ranges):
                cv = jnp.where(iota == q, counters[q], cv)
            cvec[...] = cv
            pltpu.sync_copy(cvec, counts_out.at[c])

    return pl.kernel(
        body,
        out_type=(
            jax.ShapeDtypeStruct((_NC, n_ranges, G, _AU), jnp.int32),
            jax.ShapeDtypeStruct((_NC, n_ranges, G, _AU), jnp.int32),
            jax.ShapeDtypeStruct((_NC, _L), jnp.int32),
        ),
        mesh=mesh,
        compiler_params=pltpu.CompilerParams(needs_layout_passes=False),
        scratch_types=(
            [pltpu.SMEM((_L,), jnp.int32),
             pltpu.VMEM((_B,), jnp.int32),
             pltpu.VMEM((_B,), jnp.int32),
             pltpu.VMEM((n_ranges * 2 * _AU,), jnp.int32),
             pltpu.VMEM((n_ranges * 2 * _AU,), jnp.int32),
             pltpu.VMEM((_L,), jnp.int32),
             pltpu.VMEM((2 * _L,), jnp.int32),
             pltpu.VMEM((2 * _L,), jnp.int32)]))


def _group_count(cnts_v, r, q):
    iota = jnp.arange(_L, dtype=jnp.int32)
    return jnp.max(jnp.where(iota == q, cnts_v[r], 0))


@functools.lru_cache(maxsize=None)
def _agg_kernel(n_src, n_dst, E):
    n_ranges, rng, rpt = _ranges(n_dst)
    mesh = plsc.VectorSubcoreMesh(core_axis_name="c", subcore_axis_name="s")

    def body(x_hbm, psrc, pdst, counts_hbm, zeros_hbm, out_hbm,
             acc, cnts_v, idxs, idxd, idxd2, rows, sem, sem2):
        c = lax.axis_index("c")
        s = lax.axis_index("s")
        pltpu.sync_copy(counts_hbm, cnts_v)

        def one_pass(q):
            lo = q * rng
            pltpu.sync_copy(zeros_hbm.at[pl.ds(0, rpt)],
                            acc.at[pl.ds(s * rpt, rpt)])
            plsc.subcore_barrier()
            for r in range(_NC):
                gq = _group_count(cnts_v, r, q)
                npt = (gq + _NS - 1) // _NS
                n_i = jnp.clip(gq - s * npt, 0, npt)

                def gblk(i, carry):
                    ci = s * npt + i
                    pltpu.sync_copy(psrc.at[r, q, ci], idxs)
                    cp = pltpu.async_copy(x_hbm.at[idxs], rows, sem)
                    pltpu.sync_copy(pdst.at[r, q, ci], idxd)
                    _remap(idxd, idxd2, lo, rng, _AU)
                    cp.wait()
                    pltpu.sync_copy(rows, acc.at[idxd2], add=True)
                    return carry

                lax.fori_loop(0, n_i, gblk, 0)
            plsc.subcore_barrier()
            pltpu.sync_copy(acc.at[pl.ds(s * rpt, rpt)],
                            out_hbm.at[pl.ds(lo + s * rpt, rpt)])
            plsc.subcore_barrier()

        for half in range(_NC):
            @pl.when(c == half)
            def _():
                for q in range(half, n_ranges, _NC):
                    one_pass(q)

    return pl.kernel(
        body,
        out_type=jax.ShapeDtypeStruct((n_ranges * rng, _D), jnp.float32),
        mesh=mesh,
        compiler_params=pltpu.CompilerParams(needs_layout_passes=False),
        scratch_types=[
            pltpu.VMEM_SHARED((rng + 128, _D), jnp.float32),
            pltpu.VMEM((_NC, _L), jnp.int32),
            pltpu.VMEM((_AU,), jnp.int32),
            pltpu.VMEM((_AU,), jnp.int32),
            pltpu.VMEM((_AU,), jnp.int32),
            pltpu.VMEM((_AU, _D), jnp.float32),
            pltpu.SemaphoreType.DMA,
            pltpu.SemaphoreType.DMA,
        ])


@functools.lru_cache(maxsize=None)
def _cnt_kernel(n_dst, E):
    n_ranges, rng, rpt = _ranges(n_dst)
    mesh = plsc.VectorSubcoreMesh(core_axis_name="c", subcore_axis_name="s")

    def body(pdst, counts_hbm, zeros_hbm, ones_hbm, out_hbm,
             acc, cnts_v, idxd, idxd2, ones_v, sem):
        c = lax.axis_index("c")
        s = lax.axis_index("s")
        pltpu.sync_copy(counts_hbm, cnts_v)
        pltpu.sync_copy(ones_hbm, ones_v)

        def one_pass(q):
            lo = q * rng
            pltpu.sync_copy(zeros_hbm.at[pl.ds(0, rpt)],
                            acc.at[pl.ds(s * rpt, rpt)])
            plsc.subcore_barrier()
            for r in range(_NC):
                gq = _group_count(cnts_v, r, q)
                npt = (gq + _NS - 1) // _NS
                n_i = jnp.clip(gq - s * npt, 0, npt)

                def gblk(i, carry):
                    ci = s * npt + i
                    pltpu.sync_copy(pdst.at[r, q, ci], idxd)
                    _remap(idxd, idxd2, lo, rng, _AU)
                    pltpu.sync_copy(ones_v, acc.at[idxd2], add=True)
                    return carry

                lax.fori_loop(0, n_i, gblk, 0)
            plsc.subcore_barrier()
            pltpu.sync_copy(acc.at[pl.ds(s * rpt, rpt)],
                            out_hbm.at[pl.ds(lo + s * rpt, rpt)])
            plsc.subcore_barrier()

        for half in range(_NC):
            @pl.when(c == half)
            def _():
                for q in range(half, n_ranges, _NC):
                    one_pass(q)

    return pl.kernel(
        body,
        out_type=jax.ShapeDtypeStruct((n_ranges * rng, _D), jnp.float32),
        mesh=mesh,
        compiler_params=pltpu.CompilerParams(needs_layout_passes=False),
        scratch_types=[
            pltpu.VMEM_SHARED((rng + 128, _D), jnp.float32),
            pltpu.VMEM((_NC, _L), jnp.int32),
            pltpu.VMEM((_AU,), jnp.int32),
            pltpu.VMEM((_AU,), jnp.int32),
            pltpu.VMEM((_AU, _D), jnp.float32),
            pltpu.SemaphoreType.DMA,
        ])


_R = 1000


@functools.lru_cache(maxsize=None)
def _combine_kernel(n, n_et):
    def body(*refs):
        aggs = refs[0:n_et]
        recips = refs[n_et:2 * n_et]
        x_ref = refs[2 * n_et]
        wls = refs[2 * n_et + 1:3 * n_et + 1]
        wr_ref = refs[3 * n_et + 1]
        b_ref = refs[3 * n_et + 2]
        out_ref = refs[3 * n_et + 3]
        acc = jnp.dot(x_ref[...], wr_ref[...],
                      preferred_element_type=jnp.float32) + b_ref[...]
        for a, r, w in zip(aggs, recips, wls):
            acc = acc + jnp.dot(a[...] * r[...], w[...],
                                preferred_element_type=jnp.float32)
        out_ref[...] = jnp.maximum(acc, 0.0)

    row_spec = pl.BlockSpec((_R, _D), lambda i: (i, 0))
    one_spec = pl.BlockSpec((_R, 1), lambda i: (i, 0))
    w_spec = pl.BlockSpec((_D, _D), lambda i: (0, 0))
    b_spec = pl.BlockSpec((1, _D), lambda i: (0, 0))
    in_specs = ([row_spec] * n_et + [one_spec] * n_et + [row_spec]
                + [w_spec] * n_et + [w_spec, b_spec])
    return pl.pallas_call(
        body,
        grid=(n // _R,),
        in_specs=in_specs,
        out_specs=row_spec,
        out_shape=jax.ShapeDtypeStruct((n, _D), jnp.float32),
    )


def _pad_blocks(a, nb, fill):
    total = _NS * nb * _B
    a = jnp.concatenate(
        [a, jnp.full((total - a.shape[0],), fill, jnp.int32)])
    return a.reshape(_NS, nb, _B)


def kernel(params, edges):
    parts = {}
    cnt_recip = {}
    for (s, r, d) in _ETS:
        k = _ekey(s, r, d)
        e = edges[k]
        E = e.shape[1]
        n_dst = _NODE[d]
        _, _, rpt = _ranges(n_dst)
        nb = -(-E // (_NS * _B))
        nb += nb % 2
        srcb = _pad_blocks(e[0], nb, 0)
        dstb = _pad_blocks(e[1], nb, n_dst)
        psrc, pdst, gcnt = _part_kernel(n_dst, nb, E)(srcb, dstb)
        parts[k] = (psrc, pdst, gcnt, E)
        zeros = jnp.zeros((rpt, _D), jnp.float32)
        ones = jnp.ones((_AU, _D), jnp.float32)
        cnt = _cnt_kernel(n_dst, E)(pdst, gcnt, zeros, ones)[:n_dst, 0]
        cnt_recip[k] = (1.0 / jnp.maximum(cnt, 1.0)).reshape(n_dst, 1)

    x = {nt: params["emb"][nt] for nt in _NODE}
    for l in range(2):
        lp = params["l" + str(l)]
        aggs = {}
        for (s, r, d) in _ETS:
            k = _ekey(s, r, d)
            psrc, pdst, gcnt, E = parts[k]
            n_dst = _NODE[d]
            _, _, rpt = _ranges(n_dst)
            zeros = jnp.zeros((rpt, _D), jnp.float32)
            aggs[k] = _agg_kernel(_NODE[s], n_dst, E)(
                x[s], psrc, pdst, gcnt, zeros)
        new_x = {}
        for nt in _NODE:
            ets = [(s, r, d) for (s, r, d) in _ETS if d == nt]
            n_et = len(ets)
            ks = [_ekey(*et) for et in ets]
            wr = sum(lp[k]["W_r"] for k in ks) / n_et
            bb = (sum(lp[k]["b_l"] for k in ks) / n_et).reshape(1, _D)
            args = ([aggs[k] for k in ks]
                    + [cnt_recip[k] for k in ks]
                    + [x[nt]] + [lp[k]["W_l"] / n_et for k in ks]
                    + [wr, bb])
            new_x[nt] = _combine_kernel(_NODE[nt], n_et)(*args)
        x = new_x
    return (x["drug"], x["disease"], x["gene"])
